# baseline (speedup 1.0000x reference)
import functools

import jax
import jax.numpy as jnp
from jax import lax
from jax.experimental import pallas as pl
from jax.experimental.pallas import tpu as pltpu
from jax.experimental.pallas import tpu_sc as plsc

F = 81
FP = 128
BS = 128
TBS = 80
NW = 32
CADD = 6


def _mm_nt_body(b_ref, x_ref, o_ref):
    o_ref[...] = lax.dot_general(
        b_ref[...], x_ref[...], (((0,), (0,)), ((), ())),
        preferred_element_type=jnp.float32)


def _mm_nt(b, x, block_cols):
    k, n = x.shape
    m = b.shape[1]
    return pl.pallas_call(
        _mm_nt_body,
        grid=(pl.cdiv(n, block_cols),),
        in_specs=[pl.BlockSpec((k, m), lambda i: (0, 0)),
                  pl.BlockSpec((k, block_cols), lambda i: (0, i))],
        out_specs=pl.BlockSpec((m, block_cols), lambda i: (0, i)),
        out_shape=jax.ShapeDtypeStruct((m, n), jnp.float32),
    )(b, x)


def _mm_body(x_ref, b_ref, o_ref):
    o_ref[...] = jnp.dot(x_ref[...], b_ref[...],
                         preferred_element_type=jnp.float32)


def _mm(x, b, block_rows):
    m, k = x.shape
    n = b.shape[1]
    return pl.pallas_call(
        _mm_body,
        grid=(m // block_rows,),
        in_specs=[pl.BlockSpec((block_rows, k), lambda i: (i, 0)),
                  pl.BlockSpec((k, n), lambda i: (0, 0))],
        out_specs=pl.BlockSpec((block_rows, n), lambda i: (i, 0)),
        out_shape=jax.ShapeDtypeStruct((m, n), jnp.float32),
    )(x, b)


def _sc_fix(out_ref, g, sel2d, flags_t, nbw, n_dummy, nb_tail, e0_tail):
    mesh = plsc.VectorSubcoreMesh(core_axis_name="c", subcore_axis_name="s")

    @functools.partial(
        pl.kernel,
        out_type=(),
        mesh=mesh,
        scratch_types=[
            pltpu.VMEM((1, nbw), jnp.int32),
            pltpu.VMEM((1, BS), jnp.int32),
            pltpu.VMEM((BS,), jnp.int32),
            pltpu.VMEM((BS, FP), jnp.float32),
            pltpu.VMEM((96, BS), jnp.float32),
            pltpu.VMEM((1, BS), jnp.int32),
            pltpu.VMEM((TBS,), jnp.int32),
            pltpu.VMEM((TBS, FP), jnp.float32),
            pltpu.VMEM((96, TBS), jnp.float32),
            pltpu.SemaphoreType.DMA,
        ],
    )
    def fix(out_hbm, g_hbm, sel_hbm, flags_hbm, flags_v, selb_v, idx_v,
            gbuf, obuf, selt_v, idxt_v, gtbuf, otbuf, sem):
        wid = lax.axis_index("s") * 2 + lax.axis_index("c")
        pltpu.sync_copy(flags_hbm.at[pl.ds(wid, 1)], flags_v)
        iota = lax.iota(jnp.int32, 16)
        perm = {k: iota ^ k for k in (8, 4, 2, 1)}

        def dg(v, idx):
            return lax.gather(
                v, idx[:, None],
                lax.GatherDimensionNumbers(
                    offset_dims=(), collapsed_slice_dims=(0,),
                    start_index_map=(0,)),
                (1,), mode=lax.GatherScatterMode.PROMISE_IN_BOUNDS)

        def accum(n_jc, gb, ob):
            def jc_body(jc, _):
                j0 = jc * 16

                def c_body(c, _):
                    f0 = c * 16
                    vs = [gb[j0 + jj, pl.ds(f0, 16)] for jj in range(16)]
                    for k in (8, 4, 2, 1):
                        old = vs
                        vs = []
                        for j in range(16):
                            u = dg(old[j ^ k], perm[k])
                            keep = (iota & k) == (j & k)
                            vs.append(jnp.where(keep, old[j], u))
                    for f in range(16):
                        ob[f0 + f, pl.ds(j0, 16)] = (
                            ob[f0 + f, pl.ds(j0, 16)] + vs[f])
                    return 0

                lax.fori_loop(0, CADD, c_body, 0)
                return 0

            lax.fori_loop(0, n_jc, jc_body, 0)

        def chunk_body(cc, _):
            fv = flags_v[0, pl.ds(16 * cc, 16)]
            for i in range(16):

                @pl.when(fv[i] > 0)
                def _():
                    b_id = (16 * cc + i) * NW + wid
                    e0 = pl.multiple_of(b_id * BS, BS)
                    pltpu.sync_copy(sel_hbm.at[pl.ds(b_id, 1), :], selb_v)
                    for kk in range(BS // 16):
                        sv = selb_v[0, pl.ds(16 * kk, 16)]
                        idx_v[pl.ds(16 * kk, 16)] = jnp.where(
                            sv >= 0, sv, n_dummy)
                    pltpu.async_copy(g_hbm.at[idx_v], gbuf, sem).wait()
                    pltpu.sync_copy(out_hbm.at[:, pl.ds(e0, BS)],
                                    obuf.at[pl.ds(0, F), :])
                    accum(BS // 16, gbuf, obuf)
                    pltpu.sync_copy(obuf.at[pl.ds(0, F), :],
                                    out_hbm.at[:, pl.ds(e0, BS)])
            return 0

        lax.fori_loop(0, nbw // 16, chunk_body, 0)

        @pl.when(wid == 0)
        def _():
            pltpu.sync_copy(sel_hbm.at[pl.ds(nb_tail, 1), :], selt_v)
            for kk in range(TBS // 16):
                sv = selt_v[0, pl.ds(16 * kk, 16)]
                idxt_v[pl.ds(16 * kk, 16)] = jnp.where(sv >= 0, sv, n_dummy)
            pltpu.async_copy(g_hbm.at[idxt_v], gtbuf, sem).wait()
            pltpu.sync_copy(out_hbm.at[:, pl.ds(e0_tail, TBS)],
                            otbuf.at[pl.ds(0, F), :])
            accum(TBS // 16, gtbuf, otbuf)
            pltpu.sync_copy(otbuf.at[pl.ds(0, F), :],
                            out_hbm.at[:, pl.ds(e0_tail, TBS)])

    fix(out_ref, g, sel2d, flags_t)


def kernel(feature_node, feature_edge, edge_index, S_hop,
           cg_00, cg_01, cg_02, cg_10, cg_11, cg_12, cg_20, cg_21, cg_22):
    E = feature_edge.shape[0]
    N = feature_node.shape[0]
    cgs = [cg_00, cg_01, cg_02, cg_10, cg_11, cg_12, cg_20, cg_21, cg_22]

    b_mat = jnp.zeros((F, F), jnp.float32)
    o = 0
    for cg in cgs:
        m1, m2, sb = cg.shape
        b_mat = b_mat.at[o:o + sb, o:o + sb].set(cg.reshape(m1 * m2, sb).T)
        o += sb
    b_pad = jnp.zeros((F, FP), jnp.float32).at[:, :F].set(b_mat)

    src = edge_index[0]
    dst = edge_index[1]
    normsq = jnp.sum(S_hop.astype(jnp.float32) ** 2, axis=-1)
    sel = jnp.where((src == dst) & (normsq < 1e-12), src, -1)
    sel = sel.astype(jnp.int32)

    nb_main = E // BS
    e0_tail = nb_main * BS
    nbw = 16 * (-(-nb_main // (NW * 16)))
    nb_pad = nbw * NW
    sel_pad = jnp.concatenate(
        [sel, jnp.full((nb_pad * BS - E,), -1, jnp.int32)])
    sel2d = sel_pad.reshape(nb_pad, BS)
    bhit = jnp.any(sel2d >= 0, axis=1)
    bhit = bhit & (jnp.arange(nb_pad) < nb_main)
    flags_t = bhit.reshape(nbw, NW).T.astype(jnp.int32)

    n_pad = N + 16
    fn_pad = jnp.zeros((n_pad, F), jnp.float32).at[:N].set(feature_node)

    g = _mm(fn_pad, b_pad, block_rows=n_pad // 6)
    out_t = _mm_nt(b_mat, feature_edge.T, block_cols=4096)

    ref = jax.new_ref(out_t)
    _sc_fix(ref, g, sel2d, flags_t, nbw, N, nb_main, e0_tail)
    return ref[...].T


# speedup vs baseline: 1.0298x; 1.0298x over previous
import functools

import jax
import jax.numpy as jnp
from jax import lax
from jax.experimental import pallas as pl
from jax.experimental.pallas import tpu as pltpu
from jax.experimental.pallas import tpu_sc as plsc

F = 81
FP = 128
BS = 128
TBS = 80
NW = 32
CADD = 6


def _mm_nt_body(b_ref, x_ref, o_ref):
    o_ref[...] = lax.dot_general(
        b_ref[...], x_ref[...], (((0,), (0,)), ((), ())),
        preferred_element_type=jnp.float32)


def _mm_nt(b, x, block_cols):
    k, n = x.shape
    m = b.shape[1]
    return pl.pallas_call(
        _mm_nt_body,
        grid=(pl.cdiv(n, block_cols),),
        in_specs=[pl.BlockSpec((k, m), lambda i: (0, 0)),
                  pl.BlockSpec((k, block_cols), lambda i: (0, i))],
        out_specs=pl.BlockSpec((m, block_cols), lambda i: (0, i)),
        out_shape=jax.ShapeDtypeStruct((m, n), jnp.float32),
    )(b, x)


def _mm_body(x_ref, b_ref, o_ref):
    o_ref[...] = jnp.dot(x_ref[...], b_ref[...],
                         preferred_element_type=jnp.float32)


def _mm(x, b, block_rows):
    m, k = x.shape
    n = b.shape[1]
    return pl.pallas_call(
        _mm_body,
        grid=(m // block_rows,),
        in_specs=[pl.BlockSpec((block_rows, k), lambda i: (i, 0)),
                  pl.BlockSpec((k, n), lambda i: (0, 0))],
        out_specs=pl.BlockSpec((block_rows, n), lambda i: (i, 0)),
        out_shape=jax.ShapeDtypeStruct((m, n), jnp.float32),
    )(x, b)


def _sc_fix(out_ref, g, sel2d, flags_t, nbw, n_dummy, nb_tail, e0_tail):
    mesh = plsc.VectorSubcoreMesh(core_axis_name="c", subcore_axis_name="s")

    @functools.partial(
        pl.kernel,
        out_type=(),
        mesh=mesh,
        scratch_types=[
            pltpu.VMEM((1, nbw), jnp.int32),
            pltpu.VMEM((1, BS), jnp.int32),
            pltpu.VMEM((BS,), jnp.int32),
            pltpu.VMEM((BS, FP), jnp.float32),
            pltpu.VMEM((F, BS), jnp.float32),
            pltpu.VMEM((1, BS), jnp.int32),
            pltpu.VMEM((TBS,), jnp.int32),
            pltpu.VMEM((TBS, FP), jnp.float32),
            pltpu.VMEM((F, TBS), jnp.float32),
            pltpu.SemaphoreType.DMA,
            pltpu.SemaphoreType.DMA,
        ],
    )
    def fix(out_hbm, g_hbm, sel_hbm, flags_hbm, flags_v, selb_v, idx_v,
            gbuf, obuf, selt_v, idxt_v, gtbuf, otbuf, sem, sem2):
        wid = lax.axis_index("s") * 2 + lax.axis_index("c")
        pltpu.sync_copy(flags_hbm.at[pl.ds(wid, 1)], flags_v)
        iota = lax.iota(jnp.int32, 16)
        perm = {k: iota ^ k for k in (8, 4, 2, 1)}

        def dg(v, idx):
            return lax.gather(
                v, idx[:, None],
                lax.GatherDimensionNumbers(
                    offset_dims=(), collapsed_slice_dims=(0,),
                    start_index_map=(0,)),
                (1,), mode=lax.GatherScatterMode.PROMISE_IN_BOUNDS)

        def accum(n_jc, gb, ob):
            def jc_body(jc, _):
                j0 = jc * 16

                def c_body(c, _):
                    f0 = c * 16
                    vs = [gb[j0 + jj, pl.ds(f0, 16)] for jj in range(16)]
                    for k in (8, 4, 2, 1):
                        old = vs
                        vs = []
                        for j in range(16):
                            u = dg(old[j ^ k], perm[k])
                            keep = (iota & k) == (j & k)
                            vs.append(jnp.where(keep, old[j], u))
                    for f in range(16):
                        ob[f0 + f, pl.ds(j0, 16)] = (
                            ob[f0 + f, pl.ds(j0, 16)] + vs[f])
                    return 0

                lax.fori_loop(0, CADD - 1, c_body, 0)
                v80 = jnp.zeros((16,), jnp.float32)
                for jj in range(16):
                    v80 = jnp.where(iota == jj,
                                    gb[j0 + jj, pl.ds(80 - jj, 16)], v80)
                ob[80, pl.ds(j0, 16)] = ob[80, pl.ds(j0, 16)] + v80
                return 0

            lax.fori_loop(0, n_jc, jc_body, 0)

        def chunk_body(cc, _):
            fv = flags_v[0, pl.ds(16 * cc, 16)]
            for i in range(16):

                @pl.when(fv[i] > 0)
                def _():
                    b_id = (16 * cc + i) * NW + wid
                    e0 = pl.multiple_of(b_id * BS, BS)
                    h_o = pltpu.async_copy(
                        out_hbm.at[:, pl.ds(e0, BS)], obuf, sem2)
                    pltpu.sync_copy(sel_hbm.at[pl.ds(b_id, 1), :], selb_v)
                    for kk in range(BS // 16):
                        sv = selb_v[0, pl.ds(16 * kk, 16)]
                        idx_v[pl.ds(16 * kk, 16)] = jnp.where(
                            sv >= 0, sv, n_dummy)
                    pltpu.async_copy(g_hbm.at[idx_v], gbuf, sem).wait()
                    h_o.wait()
                    accum(BS // 16, gbuf, obuf)
                    pltpu.sync_copy(obuf, out_hbm.at[:, pl.ds(e0, BS)])
            return 0

        lax.fori_loop(0, nbw // 16, chunk_body, 0)

        @pl.when(wid == 0)
        def _():
            h_ot = pltpu.async_copy(
                out_hbm.at[:, pl.ds(e0_tail, TBS)], otbuf, sem2)
            pltpu.sync_copy(sel_hbm.at[pl.ds(nb_tail, 1), :], selt_v)
            for kk in range(TBS // 16):
                sv = selt_v[0, pl.ds(16 * kk, 16)]
                idxt_v[pl.ds(16 * kk, 16)] = jnp.where(sv >= 0, sv, n_dummy)
            pltpu.async_copy(g_hbm.at[idxt_v], gtbuf, sem).wait()
            h_ot.wait()
            accum(TBS // 16, gtbuf, otbuf)
            pltpu.sync_copy(otbuf, out_hbm.at[:, pl.ds(e0_tail, TBS)])

    fix(out_ref, g, sel2d, flags_t)


def kernel(feature_node, feature_edge, edge_index, S_hop,
           cg_00, cg_01, cg_02, cg_10, cg_11, cg_12, cg_20, cg_21, cg_22):
    E = feature_edge.shape[0]
    N = feature_node.shape[0]
    cgs = [cg_00, cg_01, cg_02, cg_10, cg_11, cg_12, cg_20, cg_21, cg_22]

    b_mat = jnp.zeros((F, F), jnp.float32)
    o = 0
    for cg in cgs:
        m1, m2, sb = cg.shape
        b_mat = b_mat.at[o:o + sb, o:o + sb].set(cg.reshape(m1 * m2, sb).T)
        o += sb
    b_pad = jnp.zeros((F, FP), jnp.float32).at[:, :F].set(b_mat)

    src = edge_index[0]
    dst = edge_index[1]
    normsq = jnp.sum(S_hop.astype(jnp.float32) ** 2, axis=-1)
    sel = jnp.where((src == dst) & (normsq < 1e-12), src, -1)
    sel = sel.astype(jnp.int32)

    nb_main = E // BS
    e0_tail = nb_main * BS
    nbw = 16 * (-(-nb_main // (NW * 16)))
    nb_pad = nbw * NW
    sel_pad = jnp.concatenate(
        [sel, jnp.full((nb_pad * BS - E,), -1, jnp.int32)])
    sel2d = sel_pad.reshape(nb_pad, BS)
    bhit = jnp.any(sel2d >= 0, axis=1)
    bhit = bhit & (jnp.arange(nb_pad) < nb_main)
    flags_t = bhit.reshape(nbw, NW).T.astype(jnp.int32)

    n_pad = N + 16
    fn_pad = jnp.zeros((n_pad, F), jnp.float32).at[:N].set(feature_node)

    g = _mm(fn_pad, b_pad, block_rows=n_pad // 6)
    out_t = _mm_nt(b_mat, feature_edge.T, block_cols=4096)

    ref = jax.new_ref(out_t)
    _sc_fix(ref, g, sel2d, flags_t, nbw, N, nb_main, e0_tail)
    return ref[...].T


# speedup vs baseline: 1.0739x; 1.0428x over previous
import functools

import jax
import jax.numpy as jnp
from jax import lax
from jax.experimental import pallas as pl
from jax.experimental.pallas import tpu as pltpu
from jax.experimental.pallas import tpu_sc as plsc

F = 81
FP = 128
BS = 128
TBS = 80
NW = 32
CADD = 6


def _mm_nt_body(b_ref, x_ref, o_ref):
    o_ref[...] = lax.dot_general(
        b_ref[...], x_ref[...], (((0,), (0,)), ((), ())),
        preferred_element_type=jnp.float32)


def _mm_nt(b, x, block_cols):
    k, n = x.shape
    m = b.shape[1]
    return pl.pallas_call(
        _mm_nt_body,
        grid=(pl.cdiv(n, block_cols),),
        in_specs=[pl.BlockSpec((k, m), lambda i: (0, 0)),
                  pl.BlockSpec((k, block_cols), lambda i: (0, i))],
        out_specs=pl.BlockSpec((m, block_cols), lambda i: (0, i)),
        out_shape=jax.ShapeDtypeStruct((m, n), jnp.float32),
    )(b, x)


def _g_mm(b, x, n_real, n_rows, block_rows):
    k = x.shape[0]
    n = b.shape[1]

    def body(b_ref, x_ref, o_ref):
        i = pl.program_id(0)
        res = lax.dot_general(
            x_ref[...], b_ref[...], (((0,), (0,)), ((), ())),
            preferred_element_type=jnp.float32)
        rows = i * block_rows + lax.broadcasted_iota(
            jnp.int32, res.shape, 0)
        o_ref[...] = jnp.where(rows < n_real, res, 0.0)

    return pl.pallas_call(
        body,
        grid=(pl.cdiv(n_rows, block_rows),),
        in_specs=[pl.BlockSpec((k, n), lambda i: (0, 0)),
                  pl.BlockSpec((k, block_rows), lambda i: (0, i))],
        out_specs=pl.BlockSpec((block_rows, n), lambda i: (i, 0)),
        out_shape=jax.ShapeDtypeStruct((n_rows, n), jnp.float32),
    )(b, x)


def _sc_fix(out_ref, g, sel2d, flags_t, nbw, n_dummy, nb_tail, e0_tail):
    mesh = plsc.VectorSubcoreMesh(core_axis_name="c", subcore_axis_name="s")

    @functools.partial(
        pl.kernel,
        out_type=(),
        mesh=mesh,
        scratch_types=[
            pltpu.VMEM((1, nbw), jnp.int32),
            pltpu.VMEM((1, BS), jnp.int32),
            pltpu.VMEM((BS,), jnp.int32),
            pltpu.VMEM((BS, FP), jnp.float32),
            pltpu.VMEM((F, BS), jnp.float32),
            pltpu.VMEM((1, BS), jnp.int32),
            pltpu.VMEM((TBS,), jnp.int32),
            pltpu.VMEM((TBS, FP), jnp.float32),
            pltpu.VMEM((F, TBS), jnp.float32),
            pltpu.SemaphoreType.DMA,
            pltpu.SemaphoreType.DMA,
        ],
    )
    def fix(out_hbm, g_hbm, sel_hbm, flags_hbm, flags_v, selb_v, idx_v,
            gbuf, obuf, selt_v, idxt_v, gtbuf, otbuf, sem, sem2):
        wid = lax.axis_index("s") * 2 + lax.axis_index("c")
        pltpu.sync_copy(flags_hbm.at[pl.ds(wid, 1)], flags_v)
        iota = lax.iota(jnp.int32, 16)
        perm = {k: iota ^ k for k in (8, 4, 2, 1)}

        def dg(v, idx):
            return lax.gather(
                v, idx[:, None],
                lax.GatherDimensionNumbers(
                    offset_dims=(), collapsed_slice_dims=(0,),
                    start_index_map=(0,)),
                (1,), mode=lax.GatherScatterMode.PROMISE_IN_BOUNDS)

        def accum(n_jc, gb, ob):
            def jc_body(jc, _):
                j0 = jc * 16

                def c_body(c, _):
                    f0 = c * 16
                    vs = [gb[j0 + jj, pl.ds(f0, 16)] for jj in range(16)]
                    for k in (8, 4, 2, 1):
                        old = vs
                        vs = []
                        for j in range(16):
                            u = dg(old[j ^ k], perm[k])
                            keep = (iota & k) == (j & k)
                            vs.append(jnp.where(keep, old[j], u))
                    for f in range(16):
                        ob[f0 + f, pl.ds(j0, 16)] = (
                            ob[f0 + f, pl.ds(j0, 16)] + vs[f])
                    return 0

                lax.fori_loop(0, CADD - 1, c_body, 0)
                v80 = jnp.zeros((16,), jnp.float32)
                for jj in range(16):
                    v80 = jnp.where(iota == jj,
                                    gb[j0 + jj, pl.ds(80 - jj, 16)], v80)
                ob[80, pl.ds(j0, 16)] = ob[80, pl.ds(j0, 16)] + v80
                return 0

            lax.fori_loop(0, n_jc, jc_body, 0)

        def chunk_body(cc, _):
            fv = flags_v[0, pl.ds(16 * cc, 16)]
            for i in range(16):

                @pl.when(fv[i] > 0)
                def _():
                    b_id = (16 * cc + i) * NW + wid
                    e0 = pl.multiple_of(b_id * BS, BS)
                    h_o = pltpu.async_copy(
                        out_hbm.at[:, pl.ds(e0, BS)], obuf, sem2)
                    pltpu.sync_copy(sel_hbm.at[pl.ds(b_id, 1), :], selb_v)
                    for kk in range(BS // 16):
                        sv = selb_v[0, pl.ds(16 * kk, 16)]
                        idx_v[pl.ds(16 * kk, 16)] = jnp.where(
                            sv >= 0, sv, n_dummy)
                    pltpu.async_copy(g_hbm.at[idx_v], gbuf, sem).wait()
                    h_o.wait()
                    accum(BS // 16, gbuf, obuf)
                    pltpu.sync_copy(obuf, out_hbm.at[:, pl.ds(e0, BS)])
            return 0

        lax.fori_loop(0, nbw // 16, chunk_body, 0)

        @pl.when(wid == 0)
        def _():
            h_ot = pltpu.async_copy(
                out_hbm.at[:, pl.ds(e0_tail, TBS)], otbuf, sem2)
            pltpu.sync_copy(sel_hbm.at[pl.ds(nb_tail, 1), :], selt_v)
            for kk in range(TBS // 16):
                sv = selt_v[0, pl.ds(16 * kk, 16)]
                idxt_v[pl.ds(16 * kk, 16)] = jnp.where(sv >= 0, sv, n_dummy)
            pltpu.async_copy(g_hbm.at[idxt_v], gtbuf, sem).wait()
            h_ot.wait()
            accum(TBS // 16, gtbuf, otbuf)
            pltpu.sync_copy(otbuf, out_hbm.at[:, pl.ds(e0_tail, TBS)])

    fix(out_ref, g, sel2d, flags_t)


def kernel(feature_node, feature_edge, edge_index, S_hop,
           cg_00, cg_01, cg_02, cg_10, cg_11, cg_12, cg_20, cg_21, cg_22):
    E = feature_edge.shape[0]
    N = feature_node.shape[0]
    cgs = [cg_00, cg_01, cg_02, cg_10, cg_11, cg_12, cg_20, cg_21, cg_22]

    b_mat = jnp.zeros((F, F), jnp.float32)
    o = 0
    for cg in cgs:
        m1, m2, sb = cg.shape
        b_mat = b_mat.at[o:o + sb, o:o + sb].set(cg.reshape(m1 * m2, sb).T)
        o += sb
    b_pad = jnp.zeros((F, FP), jnp.float32).at[:, :F].set(b_mat)

    src = edge_index[0]
    dst = edge_index[1]
    normsq = jnp.sum(S_hop.astype(jnp.float32) ** 2, axis=-1)
    sel = jnp.where((src == dst) & (normsq < 1e-12), src, -1)
    sel = sel.astype(jnp.int32)

    nb_main = E // BS
    e0_tail = nb_main * BS
    nbw = 16 * (-(-nb_main // (NW * 16)))
    nb_pad = nbw * NW
    sel_pad = jnp.concatenate(
        [sel, jnp.full((nb_pad * BS - E,), -1, jnp.int32)])
    sel2d = sel_pad.reshape(nb_pad, BS)
    bhit = jnp.any(sel2d >= 0, axis=1)
    bhit = bhit & (jnp.arange(nb_pad) < nb_main)
    flags_t = bhit.reshape(nbw, NW).T.astype(jnp.int32)

    g = _g_mm(b_pad, feature_node.T, N, N + 16, block_rows=4096)
    out_t = _mm_nt(b_mat, feature_edge.T, block_cols=4096)

    ref = jax.new_ref(out_t)
    _sc_fix(ref, g, sel2d, flags_t, nbw, N, nb_main, e0_tail)
    return ref[...].T
